# baseline (speedup 1.0000x reference)
import functools

import jax
import jax.numpy as jnp
from jax.experimental import pallas as pl
from jax.experimental.pallas import tpu as pltpu

D_MODEL = 4096
N_EXPERTS = 64
TOP_K = 8
AUX_W = 0.01
CHUNK = 128


def _router_body(x1_ref, x2_ref, w_ref, gates_ref, idx_ref, aux_ref,
                 counts_ref, *, blk_t, n_blk, n_tokens):
    i = pl.program_id(0)
    half = D_MODEL // 2
    lt = jax.lax.dot_general(
        w_ref[:, :half], x1_ref[...],
        dimension_numbers=(((1,), (1,)), ((), ())),
        preferred_element_type=jnp.float32,
    ) + jax.lax.dot_general(
        w_ref[:, half:], x2_ref[...],
        dimension_numbers=(((1,), (1,)), ((), ())),
        preferred_element_type=jnp.float32,
    )

    sub_iota = jax.lax.broadcasted_iota(
        jnp.int32, (N_EXPERTS, CHUNK), 0).astype(jnp.float32)

    @pl.when(i == 0)
    def _init():
        counts_ref[...] = jnp.zeros_like(counts_ref)

    for c in range(blk_t // CHUNK):
        work = jax.lax.slice(lt, (0, c * CHUNK), (N_EXPERTS, (c + 1) * CHUNK))
        vals = []
        idxs = []
        for _ in range(TOP_K):
            mj = jnp.max(work, axis=0, keepdims=True)
            ij = jnp.min(jnp.where(work == mj, sub_iota, float(N_EXPERTS)),
                         axis=0, keepdims=True)
            vals.append(mj)
            idxs.append(ij)
            work = jnp.where(sub_iota == ij, -jnp.inf, work)

        v = jnp.concatenate(vals, axis=0)
        ev = jnp.exp(v - vals[0])
        g = ev / jnp.sum(ev, axis=0, keepdims=True)
        ix = jnp.concatenate(idxs, axis=0)

        gates_ref[0, pl.ds(c * CHUNK, CHUNK), :] = g.T
        idx_ref[0, pl.ds(c * CHUNK, CHUNK), :] = ix.T.astype(jnp.int32)

        counts_ref[...] += jnp.where(sub_iota == idxs[0], 1.0, 0.0)

    @pl.when(i == n_blk - 1)
    def _fin():
        freq = jnp.sum(counts_ref[...], axis=1, keepdims=True) / n_tokens
        diff = freq - (1.0 / N_EXPERTS)
        aux_ref[...] = AUX_W * N_EXPERTS * jnp.sum(diff * diff,
                                                   axis=(0, 1), keepdims=True)


def kernel(x, gate_w):
    b, s, d = x.shape
    n_tokens = b * s
    blk_t = 1024
    n_blk = n_tokens // blk_t
    xf = x.reshape(n_tokens, d)

    gates, idx, aux = pl.pallas_call(
        functools.partial(_router_body, blk_t=blk_t, n_blk=n_blk,
                          n_tokens=n_tokens),
        grid=(n_blk,),
        in_specs=[
            pl.BlockSpec((blk_t, d // 2), lambda i: (i, 0)),
            pl.BlockSpec((blk_t, d // 2), lambda i: (i, 1)),
            pl.BlockSpec((N_EXPERTS, d), lambda i: (0, 0)),
        ],
        out_specs=[
            pl.BlockSpec((1, blk_t, TOP_K),
                         lambda i: (i // (s // blk_t), i % (s // blk_t), 0)),
            pl.BlockSpec((1, blk_t, TOP_K),
                         lambda i: (i // (s // blk_t), i % (s // blk_t), 0)),
            pl.BlockSpec((1, 1), lambda i: (0, 0)),
        ],
        out_shape=[
            jax.ShapeDtypeStruct((b, s, TOP_K), jnp.float32),
            jax.ShapeDtypeStruct((b, s, TOP_K), jnp.int32),
            jax.ShapeDtypeStruct((1, 1), jnp.float32),
        ],
        scratch_shapes=[pltpu.VMEM((N_EXPERTS, CHUNK), jnp.float32)],
        compiler_params=pltpu.CompilerParams(
            vmem_limit_bytes=128 * 1024 * 1024),
    )(xf, xf, gate_w)

    return (gates, idx, aux[0, 0])


# speedup vs baseline: 1.2267x; 1.2267x over previous
import functools

import jax
import jax.numpy as jnp
from jax.experimental import pallas as pl
from jax.experimental.pallas import tpu as pltpu

D_MODEL = 4096
N_EXPERTS = 64
TOP_K = 8
AUX_W = 0.01
CHUNK = 128


def _router_body(x1_ref, x2_ref, w_ref, gates_ref, idx_ref, aux_ref,
                 counts_ref, *, blk_t, n_blk, n_tokens):
    i = pl.program_id(0)
    half = D_MODEL // 2
    lt = jax.lax.dot_general(
        w_ref[:, :half], x1_ref[...],
        dimension_numbers=(((1,), (1,)), ((), ())),
        preferred_element_type=jnp.float32,
    ) + jax.lax.dot_general(
        w_ref[:, half:], x2_ref[...],
        dimension_numbers=(((1,), (1,)), ((), ())),
        preferred_element_type=jnp.float32,
    )

    sub_iota = jax.lax.broadcasted_iota(
        jnp.int32, (N_EXPERTS, CHUNK), 0).astype(jnp.float32)

    @pl.when(i == 0)
    def _init():
        counts_ref[...] = jnp.zeros_like(counts_ref)

    for c in range(blk_t // CHUNK):
        work = jax.lax.slice(lt, (0, c * CHUNK), (N_EXPERTS, (c + 1) * CHUNK))
        vals = []
        idxs = []
        for _ in range(TOP_K):
            mj = jnp.max(work, axis=0, keepdims=True)
            ij = jnp.min(jnp.where(work == mj, sub_iota, float(N_EXPERTS)),
                         axis=0, keepdims=True)
            vals.append(mj)
            idxs.append(ij)
            work = jnp.where(sub_iota == ij, -jnp.inf, work)

        v = jnp.concatenate(vals, axis=0)
        ev = jnp.exp(v - vals[0])
        g = ev / jnp.sum(ev, axis=0, keepdims=True)
        ix = jnp.concatenate(idxs, axis=0)

        gates_ref[:, pl.ds(c * CHUNK, CHUNK)] = g
        idx_ref[:, pl.ds(c * CHUNK, CHUNK)] = ix.astype(jnp.int32)

        counts_ref[...] += jnp.where(sub_iota == idxs[0], 1.0, 0.0)

    @pl.when(i == n_blk - 1)
    def _fin():
        freq = jnp.sum(counts_ref[...], axis=1, keepdims=True) / n_tokens
        diff = freq - (1.0 / N_EXPERTS)
        aux_ref[...] = AUX_W * N_EXPERTS * jnp.sum(diff * diff,
                                                   axis=(0, 1), keepdims=True)


def kernel(x, gate_w):
    b, s, d = x.shape
    n_tokens = b * s
    blk_t = 1024
    n_blk = n_tokens // blk_t
    xf = x.reshape(n_tokens, d)

    gates, idx, aux = pl.pallas_call(
        functools.partial(_router_body, blk_t=blk_t, n_blk=n_blk,
                          n_tokens=n_tokens),
        grid=(n_blk,),
        in_specs=[
            pl.BlockSpec((blk_t, d // 2), lambda i: (i, 0)),
            pl.BlockSpec((blk_t, d // 2), lambda i: (i, 1)),
            pl.BlockSpec((N_EXPERTS, d), lambda i: (0, 0)),
        ],
        out_specs=[
            pl.BlockSpec((TOP_K, blk_t), lambda i: (0, i)),
            pl.BlockSpec((TOP_K, blk_t), lambda i: (0, i)),
            pl.BlockSpec((1, 1), lambda i: (0, 0)),
        ],
        out_shape=[
            jax.ShapeDtypeStruct((TOP_K, n_tokens), jnp.float32),
            jax.ShapeDtypeStruct((TOP_K, n_tokens), jnp.int32),
            jax.ShapeDtypeStruct((1, 1), jnp.float32),
        ],
        scratch_shapes=[pltpu.VMEM((N_EXPERTS, CHUNK), jnp.float32)],
        compiler_params=pltpu.CompilerParams(
            vmem_limit_bytes=128 * 1024 * 1024),
    )(xf, xf, gate_w)

    return (gates.T.reshape(b, s, TOP_K), idx.T.reshape(b, s, TOP_K),
            aux[0, 0])
